# baseline (speedup 1.0000x reference)
import functools
import jax
import jax.numpy as jnp
from jax import lax
from jax.experimental import pallas as pl
from jax.experimental.pallas import tpu as pltpu
from jax.experimental.pallas import tpu_sc as plsc

CLS = 64
CH = 128
WT = 512
T = 128
NW = 32
LANE = 16


def _lrelu(x):
    return jnp.where(x >= 0, x, 0.01 * x)


def _vgather(vec, idx):
    dn = lax.GatherDimensionNumbers(
        offset_dims=(), collapsed_slice_dims=(0,), start_index_map=(0,))
    return lax.gather(vec, idx[:, None], dn, (1,),
                      mode=lax.GatherScatterMode.PROMISE_IN_BOUNDS)


def _stats_body(x_ref, wcl_ref, bcl_ref, wrg_ref, brg_ref, out_ref, acc):
    step = pl.program_id(0) * pl.num_programs(1) + pl.program_id(1)

    @pl.when(step == 0)
    def _():
        acc[...] = jnp.zeros_like(acc)

    x = x_ref[0]
    ycl = jnp.dot(wcl_ref[...], x, preferred_element_type=jnp.float32) + bcl_ref[...][:, 0:1]
    yrg = jnp.dot(wrg_ref[...], x, preferred_element_type=jnp.float32) + brg_ref[...][:, 0:1]
    acc[...] += jnp.concatenate(
        [
            jnp.sum(ycl, axis=1, keepdims=True),
            jnp.sum(ycl * ycl, axis=1, keepdims=True),
            jnp.sum(yrg, axis=1, keepdims=True),
            jnp.sum(yrg * yrg, axis=1, keepdims=True),
        ],
        axis=1,
    )

    @pl.when(step == pl.num_programs(0) * pl.num_programs(1) - 1)
    def _():
        out_ref[...] = acc[...]


def _main_body(
    x_ref, wcl1_ref, sc1_ref, sh1_ref, wcl2_ref, bcl2_ref, wcl3_ref, bcl3_ref,
    wrg_ref, scr_ref, shr_ref,
    mask_ref, inds_ref, xr_ref, hist_ref,
):
    x = x_ref[0]
    wt = x.shape[1]

    y = jnp.dot(wcl1_ref[...], x, preferred_element_type=jnp.float32)
    h1 = _lrelu(y * sc1_ref[...][:, 0:1] + sh1_ref[...][:, 0:1])
    h2 = _lrelu(jnp.dot(wcl2_ref[...], h1, preferred_element_type=jnp.float32)
                + bcl2_ref[...][:, 0:1])
    logits = jnp.dot(wcl3_ref[...], h2, preferred_element_type=jnp.float32) + bcl3_ref[...][:, 0:1]
    cls = logits[0:CLS, :]
    m = jnp.max(cls, axis=0, keepdims=True)
    iota = lax.broadcasted_iota(jnp.int32, (CLS, wt), 0)
    ind = jnp.min(jnp.where(cls == m, iota, CLS), axis=0, keepdims=True)
    mask_ref[0, 0] = _lrelu(logits[CLS : CLS + 1, :])
    inds_ref[0, 0] = ind

    onehot = (lax.broadcasted_iota(jnp.int32, (CLS, wt), 0) == ind).astype(jnp.int32)
    hsum = jnp.sum(onehot, axis=1, keepdims=True)
    hist_ref[0] = jnp.transpose(hsum)

    yr = jnp.dot(wrg_ref[...], x, preferred_element_type=jnp.float32)
    xr = _lrelu(yr * scr_ref[...][:, 0:1] + shr_ref[...][:, 0:1])
    xr_ref[...] = jnp.transpose(xr)


def _rank_in_vreg(idx, lane):
    sk, sv = plsc.sort_key_val(idx, lane)
    prev = _vgather(sk, jnp.maximum(lane - 1, 0))
    change = (lane == 0) | (sk != prev)
    start = plsc.cummax(jnp.where(change, lane, 0))
    rank = lane - start
    nxt = _vgather(sk, jnp.minimum(lane + 1, 15))
    is_last = (lane == 15) | (sk != nxt)
    return sk, sv, rank, is_last


def _make_route_kernel(N, NTILES, NP, NPT, GIDPAD, mesh):
    chunk = N // NW
    nv = chunk // LANE

    @functools.partial(
        pl.kernel, mesh=mesh,
        out_type=[
            jax.ShapeDtypeStruct((NW, chunk // T, T), jnp.int32),
            jax.ShapeDtypeStruct((NP, CH), jnp.float32),
            jax.ShapeDtypeStruct((NPT,), jnp.int32),
        ],
        scratch_types=[
            pltpu.VMEM((chunk,), jnp.int32),
            pltpu.VMEM((NTILES, CLS), jnp.int32),
            pltpu.VMEM((CLS,), jnp.int32),
            pltpu.VMEM((chunk // T, T), jnp.int32),
            pltpu.VMEM((LANE,), jnp.int32),
            pltpu.VMEM((GIDPAD,), jnp.int32),
            pltpu.VMEM((T * 4, CH), jnp.float32),
            pltpu.SemaphoreType.DMA,
        ],
        compiler_params=pltpu.CompilerParams(needs_layout_passes=False),
    )
    def route(inds_hbm, hist_hbm, xr_hbm, dest_hbm, xs_hbm, gid_hbm,
              idx_v, hist_v, cnt_v, dest2d_v, tmp16_v, gid_v, rows_v, sem):
        wid = lax.axis_index("s") * 2 + lax.axis_index("c")
        lane = lax.iota(jnp.int32, LANE)
        pltpu.sync_copy(hist_hbm, hist_v)
        pltpu.sync_copy(inds_hbm.at[pl.ds(wid * chunk, chunk)], idx_v)

        tiles_per_chunk = NTILES // NW
        tot, par = [], []
        for j in range(4):
            acc = jnp.zeros((LANE,), jnp.int32)
            pacc = jnp.zeros((LANE,), jnp.int32)
            for t in range(NTILES):
                row = hist_v[t, pl.ds(j * LANE, LANE)]
                acc = acc + row
                pred = (t < wid * tiles_per_chunk).astype(jnp.int32)
                pacc = pacc + row * pred
            tot.append(acc)
            par.append(pacc)

        carry = jnp.int32(0)
        tile_off = []
        for j in range(4):
            aligned = ((tot[j] + (T - 1)) >> 7) << 7
            cs = plsc.cumsum(aligned)
            off = cs - aligned + carry
            carry = carry + jnp.sum(aligned)
            cnt_v[pl.ds(j * LANE, LANE)] = off + par[j]
            tile_off.append(off >> 7)

        @pl.when(wid == 0)
        def _():
            for jv in range(GIDPAD // LANE):
                gid_v[pl.ds(jv * LANE, LANE)] = jnp.zeros((LANE,), jnp.int32)
            for j in range(4):
                sk, sv, rank, is_last = _rank_in_vreg(tile_off[j], lane)
                base = plsc.load_gather(gid_v, [sk])
                plsc.store_scatter(gid_v, [sk], base + rank + 1, mask=is_last)
            c2 = jnp.int32(0)
            for jv in range(GIDPAD // LANE):
                seg = gid_v[pl.ds(jv * LANE, LANE)]
                s = jnp.sum(seg)
                gid_v[pl.ds(jv * LANE, LANE)] = plsc.cumsum(seg) + c2 - 1
                c2 = c2 + s
            pltpu.sync_copy(gid_v.at[pl.ds(0, NPT)], gid_hbm)

        for v in range(nv):
            idx = idx_v[pl.ds(v * LANE, LANE)]
            sk, sv, rank, is_last = _rank_in_vreg(idx, lane)
            base = plsc.load_gather(cnt_v, [sk])
            plsc.store_scatter(cnt_v, [sk], base + rank + 1, mask=is_last)
            plsc.store_scatter(tmp16_v, [sv], base + rank)
            dest2d_v[v // 8, pl.ds((v % 8) * LANE, LANE)] = tmp16_v[...]

        pltpu.sync_copy(dest2d_v, dest_hbm.at[wid])

        for sub in range(chunk // (T * 4)):
            pltpu.sync_copy(
                xr_hbm.at[pl.ds(wid * chunk + sub * T * 4, T * 4)], rows_v)
            descs = [
                pltpu.async_copy(
                    rows_v.at[pl.ds(j * T, T)],
                    xs_hbm.at[dest2d_v.at[sub * 4 + j]],
                    sem,
                )
                for j in range(4)
            ]
            for d in descs:
                d.wait()

    return route


def _make_gather_kernel(N, NP, mesh):
    chunk = N // NW
    nv = chunk // LANE

    @functools.partial(
        pl.kernel, mesh=mesh,
        out_type=jax.ShapeDtypeStruct((NW, chunk // T, T), jnp.float32),
        scratch_types=[
            pltpu.VMEM((NP,), jnp.float32),
            pltpu.VMEM((chunk // T, T), jnp.int32),
            pltpu.VMEM((chunk // T, T), jnp.float32),
        ],
        compiler_params=pltpu.CompilerParams(needs_layout_passes=False),
    )
    def gather_back(val_hbm, dest_hbm, out_hbm, val_v, dv, ov):
        wid = lax.axis_index("s") * 2 + lax.axis_index("c")
        pltpu.sync_copy(val_hbm, val_v)
        pltpu.sync_copy(dest_hbm.at[wid], dv)
        for v in range(nv):
            d = dv[v // 8, pl.ds((v % 8) * LANE, LANE)]
            ov[v // 8, pl.ds((v % 8) * LANE, LANE)] = plsc.load_gather(val_v, [d])
        pltpu.sync_copy(ov, out_hbm.at[wid])

    return gather_back


def _group_body(gid_ref, xs_ref, w2_ref, b2_ref, w3_ref, b3_ref, out_ref):
    i = pl.program_id(0)
    g = gid_ref[i]
    x = xs_ref[...]
    z = jnp.dot(x, w2_ref[0], preferred_element_type=jnp.float32)
    z = _lrelu(z + b2_ref[0])
    w3t = jnp.transpose(w3_ref[0])
    y = jnp.sum(z * w3t, axis=1, keepdims=True) + b3_ref[0]
    val = (g.astype(jnp.float32) + y) * (1.0 / CLS)
    out_ref[0] = jnp.transpose(val)


def kernel(x_in, W_cl1, b_cl1, g1, be1, W_cl2, b_cl2, W_cl3, b_cl3,
           W_reg1, b_reg1, gr, br, W_cm2, b_cm2, W_cm3, b_cm3):
    B, Cin, H, Wd = x_in.shape
    N = B * H * Wd
    NTILES = N // WT
    NP = N + CLS * T
    NPT = NP // T
    GIDPAD = ((NPT + 1 + LANE - 1) // LANE) * LANE
    x3 = x_in.reshape(B, Cin, H * Wd)
    grid = (B, (H * Wd) // WT)

    def _col(v):
        return v.reshape(-1, 1)

    stats = pl.pallas_call(
        _stats_body,
        grid=grid,
        in_specs=[
            pl.BlockSpec((1, Cin, WT), lambda b, w: (b, 0, w)),
            pl.BlockSpec((CH, Cin), lambda b, w: (0, 0)),
            pl.BlockSpec((CH, 1), lambda b, w: (0, 0)),
            pl.BlockSpec((CH, Cin), lambda b, w: (0, 0)),
            pl.BlockSpec((CH, 1), lambda b, w: (0, 0)),
        ],
        out_specs=pl.BlockSpec((CH, 4), lambda b, w: (0, 0)),
        out_shape=jax.ShapeDtypeStruct((CH, 4), jnp.float32),
        scratch_shapes=[pltpu.VMEM((CH, 4), jnp.float32)],
    )(x3, W_cl1, _col(b_cl1), W_reg1, _col(b_reg1))

    eps = 1e-5
    n = jnp.float32(N)
    mean_cl, msq_cl = stats[:, 0] / n, stats[:, 1] / n
    mean_rg, msq_rg = stats[:, 2] / n, stats[:, 3] / n
    sc1 = g1 / jnp.sqrt(msq_cl - mean_cl * mean_cl + eps)
    sh1 = be1 - mean_cl * sc1 + b_cl1 * sc1
    scr = gr / jnp.sqrt(msq_rg - mean_rg * mean_rg + eps)
    shr = br - mean_rg * scr + b_reg1 * scr

    Wcl3p = jnp.zeros((CH, CH), jnp.float32).at[: CLS + 1, :].set(W_cl3)
    bcl3p = jnp.zeros((CH,), jnp.float32).at[: CLS + 1].set(b_cl3)

    wpt = Wd // WT
    mask4d, inds4d, xr_tm, hist_t = pl.pallas_call(
        _main_body,
        grid=grid,
        in_specs=[
            pl.BlockSpec((1, Cin, WT), lambda b, w: (b, 0, w)),
            pl.BlockSpec((CH, Cin), lambda b, w: (0, 0)),
            pl.BlockSpec((CH, 1), lambda b, w: (0, 0)),
            pl.BlockSpec((CH, 1), lambda b, w: (0, 0)),
            pl.BlockSpec((CH, CH), lambda b, w: (0, 0)),
            pl.BlockSpec((CH, 1), lambda b, w: (0, 0)),
            pl.BlockSpec((CH, CH), lambda b, w: (0, 0)),
            pl.BlockSpec((CH, 1), lambda b, w: (0, 0)),
            pl.BlockSpec((CH, Cin), lambda b, w: (0, 0)),
            pl.BlockSpec((CH, 1), lambda b, w: (0, 0)),
            pl.BlockSpec((CH, 1), lambda b, w: (0, 0)),
        ],
        out_specs=[
            pl.BlockSpec((1, 1, 1, WT), lambda b, w: (b, 0, 0, w)),
            pl.BlockSpec((1, 1, 1, WT), lambda b, w: (b, 0, 0, w)),
            pl.BlockSpec((WT, CH), lambda b, w, _wpt=wpt: (b * _wpt + w, 0)),
            pl.BlockSpec((1, 1, CLS), lambda b, w, _wpt=wpt: (b * _wpt + w, 0, 0)),
        ],
        out_shape=[
            jax.ShapeDtypeStruct((B, 1, H, Wd), jnp.float32),
            jax.ShapeDtypeStruct((B, 1, H, Wd), jnp.int32),
            jax.ShapeDtypeStruct((N, CH), jnp.float32),
            jax.ShapeDtypeStruct((NTILES, 1, CLS), jnp.int32),
        ],
    )(x3, W_cl1, _col(sc1), _col(sh1), W_cl2, _col(b_cl2), Wcl3p, _col(bcl3p),
      W_reg1, _col(scr), _col(shr))

    mesh = plsc.VectorSubcoreMesh(core_axis_name="c", subcore_axis_name="s",
                                  num_cores=2, num_subcores=16)
    route = _make_route_kernel(N, NTILES, NP, NPT, GIDPAD, mesh)
    dest, xs, gid = route(inds4d.reshape(N), hist_t.reshape(NTILES, CLS), xr_tm)

    val_sorted = pl.pallas_call(
        _group_body,
        grid_spec=pltpu.PrefetchScalarGridSpec(
            num_scalar_prefetch=1,
            grid=(NPT,),
            in_specs=[
                pl.BlockSpec((T, CH), lambda i, gid_ref: (i, 0)),
                pl.BlockSpec((1, CH, 32), lambda i, gid_ref: (gid_ref[i], 0, 0)),
                pl.BlockSpec((1, 1, 32), lambda i, gid_ref: (gid_ref[i], 0, 0)),
                pl.BlockSpec((1, 32, 1), lambda i, gid_ref: (gid_ref[i], 0, 0)),
                pl.BlockSpec((1, 1, 1), lambda i, gid_ref: (gid_ref[i], 0, 0)),
            ],
            out_specs=pl.BlockSpec((1, 1, T), lambda i, gid_ref: (i, 0, 0)),
        ),
        out_shape=jax.ShapeDtypeStruct((NPT, 1, T), jnp.float32),
    )(gid, xs, W_cm2, b_cm2.reshape(CLS, 1, 32), W_cm3,
      b_cm3.reshape(CLS, 1, 1))

    gather_back = _make_gather_kernel(N, NP, mesh)
    xreal_flat = gather_back(val_sorted.reshape(NP), dest)

    x_real = xreal_flat.reshape(B, 1, H, Wd)
    return (x_real, mask4d)


# speedup vs baseline: 1.8634x; 1.8634x over previous
import functools
import jax
import jax.numpy as jnp
from jax import lax
from jax.experimental import pallas as pl
from jax.experimental.pallas import tpu as pltpu
from jax.experimental.pallas import tpu_sc as plsc

CLS = 64
CH = 128
WT = 1024
GB = 4
T = 128
NW = 32
LANE = 16


def _lrelu(x):
    return jnp.where(x >= 0, x, 0.01 * x)


def _vgather(vec, idx):
    dn = lax.GatherDimensionNumbers(
        offset_dims=(), collapsed_slice_dims=(0,), start_index_map=(0,))
    return lax.gather(vec, idx[:, None], dn, (1,),
                      mode=lax.GatherScatterMode.PROMISE_IN_BOUNDS)


def _stats_body(x_ref, wcl_ref, bcl_ref, wrg_ref, brg_ref, out_ref, acc):
    step = pl.program_id(0) * pl.num_programs(1) + pl.program_id(1)

    @pl.when(step == 0)
    def _():
        acc[...] = jnp.zeros_like(acc)

    x = x_ref[0]
    ycl = jnp.dot(wcl_ref[...], x, preferred_element_type=jnp.float32) + bcl_ref[...][:, 0:1]
    yrg = jnp.dot(wrg_ref[...], x, preferred_element_type=jnp.float32) + brg_ref[...][:, 0:1]
    acc[...] += jnp.concatenate(
        [
            jnp.sum(ycl, axis=1, keepdims=True),
            jnp.sum(ycl * ycl, axis=1, keepdims=True),
            jnp.sum(yrg, axis=1, keepdims=True),
            jnp.sum(yrg * yrg, axis=1, keepdims=True),
        ],
        axis=1,
    )

    @pl.when(step == pl.num_programs(0) * pl.num_programs(1) - 1)
    def _():
        out_ref[...] = acc[...]


def _main_body(
    x_ref, wcl1_ref, sc1_ref, sh1_ref, wcl2_ref, bcl2_ref, wcl3_ref, bcl3_ref,
    wrg_ref, scr_ref, shr_ref,
    mask_ref, inds_ref, xr_ref, hist_ref,
):
    x = x_ref[0]
    wt = x.shape[1]

    y = jnp.dot(wcl1_ref[...], x, preferred_element_type=jnp.float32)
    h1 = _lrelu(y * sc1_ref[...][:, 0:1] + sh1_ref[...][:, 0:1])
    h2 = _lrelu(jnp.dot(wcl2_ref[...], h1, preferred_element_type=jnp.float32)
                + bcl2_ref[...][:, 0:1])
    logits = jnp.dot(wcl3_ref[...], h2, preferred_element_type=jnp.float32) + bcl3_ref[...][:, 0:1]
    cls = logits[0:CLS, :]
    m = jnp.max(cls, axis=0, keepdims=True)
    iota = lax.broadcasted_iota(jnp.int32, (CLS, wt), 0)
    ind = jnp.min(jnp.where(cls == m, iota, CLS), axis=0, keepdims=True)
    mask_ref[0, 0] = _lrelu(logits[CLS : CLS + 1, :])
    inds_ref[0, 0] = ind

    onehot = (lax.broadcasted_iota(jnp.int32, (CLS, wt), 0) == ind).astype(jnp.int32)
    hsum = jnp.sum(onehot, axis=1, keepdims=True)
    hist_ref[0] = jnp.transpose(hsum)

    yr = jnp.dot(wrg_ref[...], x, preferred_element_type=jnp.float32)
    xr = _lrelu(yr * scr_ref[...][:, 0:1] + shr_ref[...][:, 0:1])
    xr_ref[...] = jnp.transpose(xr)


def _rank_in_vreg(idx, lane):
    sk, sv = plsc.sort_key_val(idx, lane)
    prev = _vgather(sk, jnp.maximum(lane - 1, 0))
    change = (lane == 0) | (sk != prev)
    start = plsc.cummax(jnp.where(change, lane, 0))
    rank = lane - start
    nxt = _vgather(sk, jnp.minimum(lane + 1, 15))
    is_last = (lane == 15) | (sk != nxt)
    return sk, sv, rank, is_last


def _make_route_kernel(N, NTILES, NP, NPT, GIDPAD, mesh):
    chunk = N // NW
    nv = chunk // LANE

    @functools.partial(
        pl.kernel, mesh=mesh,
        out_type=[
            jax.ShapeDtypeStruct((NW, chunk // T, T), jnp.int32),
            jax.ShapeDtypeStruct((NP, CH), jnp.float32),
            jax.ShapeDtypeStruct((NPT,), jnp.int32),
        ],
        scratch_types=[
            pltpu.VMEM((chunk,), jnp.int32),
            pltpu.VMEM((NTILES, CLS), jnp.int32),
            pltpu.VMEM((CLS,), jnp.int32),
            pltpu.VMEM((chunk // T, T), jnp.int32),
            pltpu.VMEM((LANE,), jnp.int32),
            pltpu.VMEM((GIDPAD,), jnp.int32),
            pltpu.VMEM((T * 4, CH), jnp.float32),
            pltpu.SemaphoreType.DMA,
        ],
        compiler_params=pltpu.CompilerParams(needs_layout_passes=False),
    )
    def route(inds_hbm, hist_hbm, xr_hbm, dest_hbm, xs_hbm, gid_hbm,
              idx_v, hist_v, cnt_v, dest2d_v, tmp16_v, gid_v, rows_v, sem):
        wid = lax.axis_index("s") * 2 + lax.axis_index("c")
        lane = lax.iota(jnp.int32, LANE)
        pltpu.sync_copy(hist_hbm, hist_v)
        pltpu.sync_copy(inds_hbm.at[pl.ds(wid * chunk, chunk)], idx_v)

        tiles_per_chunk = NTILES // NW
        tot, par = [], []
        for j in range(4):
            acc = jnp.zeros((LANE,), jnp.int32)
            pacc = jnp.zeros((LANE,), jnp.int32)
            for t in range(NTILES):
                row = hist_v[t, pl.ds(j * LANE, LANE)]
                acc = acc + row
                pred = (t < wid * tiles_per_chunk).astype(jnp.int32)
                pacc = pacc + row * pred
            tot.append(acc)
            par.append(pacc)

        carry = jnp.int32(0)
        tile_off = []
        for j in range(4):
            aligned = ((tot[j] + (T - 1)) >> 7) << 7
            cs = plsc.cumsum(aligned)
            off = cs - aligned + carry
            carry = carry + jnp.sum(aligned)
            cnt_v[pl.ds(j * LANE, LANE)] = off + par[j]
            tile_off.append(off >> 7)

        @pl.when(wid == 0)
        def _():
            for jv in range(GIDPAD // LANE):
                gid_v[pl.ds(jv * LANE, LANE)] = jnp.zeros((LANE,), jnp.int32)
            for j in range(4):
                sk, sv, rank, is_last = _rank_in_vreg(tile_off[j], lane)
                base = plsc.load_gather(gid_v, [sk])
                plsc.store_scatter(gid_v, [sk], base + rank + 1, mask=is_last)
            c2 = jnp.int32(0)
            for jv in range(GIDPAD // LANE):
                seg = gid_v[pl.ds(jv * LANE, LANE)]
                s = jnp.sum(seg)
                gid_v[pl.ds(jv * LANE, LANE)] = plsc.cumsum(seg) + c2 - 1
                c2 = c2 + s
            pltpu.sync_copy(gid_v.at[pl.ds(0, NPT)], gid_hbm)

        for v in range(nv):
            idx = idx_v[pl.ds(v * LANE, LANE)]
            sk, sv, rank, is_last = _rank_in_vreg(idx, lane)
            base = plsc.load_gather(cnt_v, [sk])
            plsc.store_scatter(cnt_v, [sk], base + rank + 1, mask=is_last)
            plsc.store_scatter(tmp16_v, [sv], base + rank)
            dest2d_v[v // 8, pl.ds((v % 8) * LANE, LANE)] = tmp16_v[...]

        pltpu.sync_copy(dest2d_v, dest_hbm.at[wid])

        for sub in range(chunk // (T * 4)):
            pltpu.sync_copy(
                xr_hbm.at[pl.ds(wid * chunk + sub * T * 4, T * 4)], rows_v)
            descs = [
                pltpu.async_copy(
                    rows_v.at[pl.ds(j * T, T)],
                    xs_hbm.at[dest2d_v.at[sub * 4 + j]],
                    sem,
                )
                for j in range(4)
            ]
            for d in descs:
                d.wait()

    return route


def _make_gather_kernel(N, NP, mesh):
    chunk = N // NW
    nv = chunk // LANE

    @functools.partial(
        pl.kernel, mesh=mesh,
        out_type=jax.ShapeDtypeStruct((NW, chunk // T, T), jnp.float32),
        scratch_types=[
            pltpu.VMEM((NP,), jnp.float32),
            pltpu.VMEM((chunk // T, T), jnp.int32),
            pltpu.VMEM((chunk // T, T), jnp.float32),
        ],
        compiler_params=pltpu.CompilerParams(needs_layout_passes=False),
    )
    def gather_back(val_hbm, dest_hbm, out_hbm, val_v, dv, ov):
        wid = lax.axis_index("s") * 2 + lax.axis_index("c")
        pltpu.sync_copy(val_hbm, val_v)
        pltpu.sync_copy(dest_hbm.at[wid], dv)
        for v in range(nv):
            d = dv[v // 8, pl.ds((v % 8) * LANE, LANE)]
            ov[v // 8, pl.ds((v % 8) * LANE, LANE)] = plsc.load_gather(val_v, [d])
        pltpu.sync_copy(ov, out_hbm.at[wid])

    return gather_back


def _group_body(gid_ref, xs_ref, *refs):
    w2_refs = refs[0:GB]
    b2_refs = refs[GB:2 * GB]
    w3_refs = refs[2 * GB:3 * GB]
    b3_refs = refs[3 * GB:4 * GB]
    out_ref = refs[4 * GB]
    i = pl.program_id(0)
    rows = []
    for k in range(GB):
        g = gid_ref[i * GB + k]
        x = xs_ref[pl.ds(k * T, T), :]
        z = jnp.dot(x, w2_refs[k][0], preferred_element_type=jnp.float32)
        z = _lrelu(z + b2_refs[k][0])
        w3t = jnp.transpose(w3_refs[k][0])
        y = jnp.sum(z * w3t, axis=1, keepdims=True) + b3_refs[k][0]
        val = (g.astype(jnp.float32) + y) * (1.0 / CLS)
        rows.append(jnp.transpose(val))
    out_ref[0] = jnp.concatenate(rows, axis=0)


def kernel(x_in, W_cl1, b_cl1, g1, be1, W_cl2, b_cl2, W_cl3, b_cl3,
           W_reg1, b_reg1, gr, br, W_cm2, b_cm2, W_cm3, b_cm3):
    B, Cin, H, Wd = x_in.shape
    N = B * H * Wd
    NTILES = N // WT
    NP = N + CLS * T
    NPT = NP // T
    GIDPAD = ((NPT + 1 + LANE - 1) // LANE) * LANE
    x3 = x_in.reshape(B, Cin, H * Wd)
    grid = (B, (H * Wd) // WT)

    def _col(v):
        return v.reshape(-1, 1)

    stats = pl.pallas_call(
        _stats_body,
        grid=grid,
        in_specs=[
            pl.BlockSpec((1, Cin, WT), lambda b, w: (b, 0, w)),
            pl.BlockSpec((CH, Cin), lambda b, w: (0, 0)),
            pl.BlockSpec((CH, 1), lambda b, w: (0, 0)),
            pl.BlockSpec((CH, Cin), lambda b, w: (0, 0)),
            pl.BlockSpec((CH, 1), lambda b, w: (0, 0)),
        ],
        out_specs=pl.BlockSpec((CH, 4), lambda b, w: (0, 0)),
        out_shape=jax.ShapeDtypeStruct((CH, 4), jnp.float32),
        scratch_shapes=[pltpu.VMEM((CH, 4), jnp.float32)],
    )(x3, W_cl1, _col(b_cl1), W_reg1, _col(b_reg1))

    eps = 1e-5
    n = jnp.float32(N)
    mean_cl, msq_cl = stats[:, 0] / n, stats[:, 1] / n
    mean_rg, msq_rg = stats[:, 2] / n, stats[:, 3] / n
    sc1 = g1 / jnp.sqrt(msq_cl - mean_cl * mean_cl + eps)
    sh1 = be1 - mean_cl * sc1 + b_cl1 * sc1
    scr = gr / jnp.sqrt(msq_rg - mean_rg * mean_rg + eps)
    shr = br - mean_rg * scr + b_reg1 * scr

    Wcl3p = jnp.zeros((CH, CH), jnp.float32).at[: CLS + 1, :].set(W_cl3)
    bcl3p = jnp.zeros((CH,), jnp.float32).at[: CLS + 1].set(b_cl3)

    wpt = Wd // WT
    mask4d, inds4d, xr_tm, hist_t = pl.pallas_call(
        _main_body,
        grid=grid,
        in_specs=[
            pl.BlockSpec((1, Cin, WT), lambda b, w: (b, 0, w)),
            pl.BlockSpec((CH, Cin), lambda b, w: (0, 0)),
            pl.BlockSpec((CH, 1), lambda b, w: (0, 0)),
            pl.BlockSpec((CH, 1), lambda b, w: (0, 0)),
            pl.BlockSpec((CH, CH), lambda b, w: (0, 0)),
            pl.BlockSpec((CH, 1), lambda b, w: (0, 0)),
            pl.BlockSpec((CH, CH), lambda b, w: (0, 0)),
            pl.BlockSpec((CH, 1), lambda b, w: (0, 0)),
            pl.BlockSpec((CH, Cin), lambda b, w: (0, 0)),
            pl.BlockSpec((CH, 1), lambda b, w: (0, 0)),
            pl.BlockSpec((CH, 1), lambda b, w: (0, 0)),
        ],
        out_specs=[
            pl.BlockSpec((1, 1, 1, WT), lambda b, w: (b, 0, 0, w)),
            pl.BlockSpec((1, 1, 1, WT), lambda b, w: (b, 0, 0, w)),
            pl.BlockSpec((WT, CH), lambda b, w, _wpt=wpt: (b * _wpt + w, 0)),
            pl.BlockSpec((1, 1, CLS), lambda b, w, _wpt=wpt: (b * _wpt + w, 0, 0)),
        ],
        out_shape=[
            jax.ShapeDtypeStruct((B, 1, H, Wd), jnp.float32),
            jax.ShapeDtypeStruct((B, 1, H, Wd), jnp.int32),
            jax.ShapeDtypeStruct((N, CH), jnp.float32),
            jax.ShapeDtypeStruct((NTILES, 1, CLS), jnp.int32),
        ],
    )(x3, W_cl1, _col(sc1), _col(sh1), W_cl2, _col(b_cl2), Wcl3p, _col(bcl3p),
      W_reg1, _col(scr), _col(shr))

    mesh = plsc.VectorSubcoreMesh(core_axis_name="c", subcore_axis_name="s",
                                  num_cores=2, num_subcores=16)
    route = _make_route_kernel(N, NTILES, NP, NPT, GIDPAD, mesh)
    dest, xs, gid = route(inds4d.reshape(N), hist_t.reshape(NTILES, CLS), xr_tm)

    def _wmap(k, shape_tail):
        return pl.BlockSpec((1,) + shape_tail,
                            lambda i, gid_ref, _k=k: (gid_ref[i * GB + _k],) + (0,) * len(shape_tail))

    val_sorted = pl.pallas_call(
        _group_body,
        grid_spec=pltpu.PrefetchScalarGridSpec(
            num_scalar_prefetch=1,
            grid=(NPT // GB,),
            in_specs=[pl.BlockSpec((GB * T, CH), lambda i, gid_ref: (i, 0))]
            + [_wmap(k, (CH, 32)) for k in range(GB)]
            + [_wmap(k, (1, 32)) for k in range(GB)]
            + [_wmap(k, (32, 1)) for k in range(GB)]
            + [_wmap(k, (1, 1)) for k in range(GB)],
            out_specs=pl.BlockSpec((1, GB, T), lambda i, gid_ref: (i, 0, 0)),
        ),
        out_shape=jax.ShapeDtypeStruct((NPT // GB, GB, T), jnp.float32),
    )(gid, xs,
      *([W_cm2] * GB),
      *([b_cm2.reshape(CLS, 1, 32)] * GB),
      *([W_cm3] * GB),
      *([b_cm3.reshape(CLS, 1, 1)] * GB))

    gather_back = _make_gather_kernel(N, NP, mesh)
    xreal_flat = gather_back(val_sorted.reshape(NP), dest)

    x_real = xreal_flat.reshape(B, 1, H, Wd)
    return (x_real, mask4d)
